# baseline (device time: 111027 ns/iter reference)
import jax
import jax.numpy as jnp
from jax import lax
from jax.experimental import pallas as pl
from jax.experimental.pallas import tpu as pltpu

N_DEV = 8
B = 2
S_PER = 256
S_FULL = N_DEV * S_PER
HQ = 4
DH = 64
D_MODEL = 512
D_QK = HQ * DH


def kernel(x, Wq, K_ext, V_ext, Wo):
    def body(x_ref, wq_ref, k_ref, v_ref, wo_ref, out_ref,
             kf_ref, vf_ref, ksend, krecv, vsend, vrecv):
        my = lax.axis_index("i")
        left = lax.rem(my + N_DEV - 1, N_DEV)
        right = lax.rem(my + 1, N_DEV)

        kc = jnp.transpose(k_ref[...].astype(jnp.bfloat16), (0, 2, 1, 3))
        vc = jnp.transpose(v_ref[...].astype(jnp.bfloat16), (0, 2, 1, 3))
        kf_ref[:, :, pl.ds(my * S_PER, S_PER), :] = kc
        vf_ref[:, :, pl.ds(my * S_PER, S_PER), :] = vc

        bar = pltpu.get_barrier_semaphore()
        for nbr in (left, right):
            pl.semaphore_signal(bar, inc=1, device_id=(nbr,),
                                device_id_type=pl.DeviceIdType.MESH)
        pl.semaphore_wait(bar, 2)

        for h in range(N_DEV - 1):
            o = lax.rem(my + N_DEV - h, N_DEV)
            sl = pl.ds(o * S_PER, S_PER)
            k_rdma = pltpu.make_async_remote_copy(
                src_ref=kf_ref.at[:, :, sl, :],
                dst_ref=kf_ref.at[:, :, sl, :],
                send_sem=ksend.at[h], recv_sem=krecv.at[h],
                device_id=(right,), device_id_type=pl.DeviceIdType.MESH,
            )
            v_rdma = pltpu.make_async_remote_copy(
                src_ref=vf_ref.at[:, :, sl, :],
                dst_ref=vf_ref.at[:, :, sl, :],
                send_sem=vsend.at[h], recv_sem=vrecv.at[h],
                device_id=(right,), device_id_type=pl.DeviceIdType.MESH,
            )
            k_rdma.start()
            v_rdma.start()
            k_rdma.wait()
            v_rdma.wait()

        qi = my * S_PER + lax.broadcasted_iota(jnp.int32, (S_PER, S_FULL), 0)
        ki = lax.broadcasted_iota(jnp.int32, (S_PER, S_FULL), 1)
        mask = (jnp.abs(qi - ki) <= 128) | (ki < 32) | (qi < 32)

        wq = wq_ref[...].astype(jnp.bfloat16)
        wo = wo_ref[...].astype(jnp.bfloat16)
        for b in range(B):
            xb = x_ref[b].astype(jnp.bfloat16)
            qb = lax.dot_general(xb, wq, (((1,), (0,)), ((), ())),
                                 preferred_element_type=jnp.float32)
            qb = qb.reshape(S_PER, HQ, DH).astype(jnp.bfloat16)
            ctx_heads = []
            for hh in range(HQ):
                qh = qb[:, hh, :]
                kh = kf_ref[b, hh]
                s = lax.dot_general(qh, kh, (((1,), (1,)), ((), ())),
                                    preferred_element_type=jnp.float32)
                s = jnp.where(mask, s * 0.125, -1e9)
                m = jnp.max(s, axis=-1, keepdims=True)
                w = jnp.exp(s - m)
                w = w / jnp.sum(w, axis=-1, keepdims=True)
                vh = vf_ref[b, hh]
                ctx_heads.append(
                    lax.dot_general(w.astype(jnp.bfloat16), vh,
                                    (((1,), (0,)), ((), ())),
                                    preferred_element_type=jnp.float32))
            ctxb = jnp.concatenate(ctx_heads, axis=-1).astype(jnp.bfloat16)
            out_ref[b] = lax.dot_general(ctxb, wo, (((1,), (0,)), ((), ())),
                                         preferred_element_type=jnp.float32)

    return pl.pallas_call(
        body,
        out_shape=jax.ShapeDtypeStruct((B, S_PER, D_MODEL), jnp.float32),
        in_specs=[pl.BlockSpec(memory_space=pltpu.VMEM)] * 5,
        out_specs=pl.BlockSpec(memory_space=pltpu.VMEM),
        scratch_shapes=[
            pltpu.VMEM((B, HQ, S_FULL, DH), jnp.bfloat16),
            pltpu.VMEM((B, HQ, S_FULL, DH), jnp.bfloat16),
            pltpu.SemaphoreType.DMA((N_DEV - 1,)),
            pltpu.SemaphoreType.DMA((N_DEV - 1,)),
            pltpu.SemaphoreType.DMA((N_DEV - 1,)),
            pltpu.SemaphoreType.DMA((N_DEV - 1,)),
        ],
        compiler_params=pltpu.CompilerParams(collective_id=0),
    )(x, Wq, K_ext, V_ext, Wo)
